# Initial kernel scaffold; baseline (speedup 1.0000x reference)
import jax
import jax.numpy as jnp
from jax.experimental import pallas as pl


def kernel(input, table):
    raise NotImplementedError("write your pallas kernel here")


# speedup vs baseline: 3.2942x; 3.2942x over previous
import functools

import jax
import jax.numpy as jnp
from jax import lax
from jax.experimental import pallas as pl
from jax.experimental.pallas import tpu as pltpu
from jax.experimental.pallas import tpu_sc as plsc

DIM = 128
NUM_CORES = 2
NUM_SUBCORES = 16
NUM_WORKERS = NUM_CORES * NUM_SUBCORES


def _make_gather(batch: int, dim: int, chunk: int):
  assert batch % NUM_WORKERS == 0
  b_per_w = batch // NUM_WORKERS
  assert b_per_w % chunk == 0 and chunk % 8 == 0
  n_chunks = b_per_w // chunk

  mesh = plsc.VectorSubcoreMesh(core_axis_name="c", subcore_axis_name="s")

  @functools.partial(
      pl.kernel,
      mesh=mesh,
      out_type=jax.ShapeDtypeStruct((batch, dim), jnp.float32),
      scratch_types=[
          pltpu.VMEM((b_per_w,), jnp.int32),
          pltpu.VMEM((chunk, dim), jnp.float32),
          pltpu.SemaphoreType.DMA,
      ],
  )
  def gather_kernel(table_hbm, idx_hbm, out_hbm, idx_v, rows_v, sem):
    wid = lax.axis_index("s") * NUM_CORES + lax.axis_index("c")
    base = wid * b_per_w
    pltpu.sync_copy(idx_hbm.at[pl.ds(base, b_per_w)], idx_v)

    def body(i, carry):
      off = i * chunk
      pltpu.async_copy(
          table_hbm.at[idx_v.at[pl.ds(off, chunk)]], rows_v, sem
      ).wait()
      pltpu.sync_copy(rows_v, out_hbm.at[pl.ds(base + off, chunk)])
      return carry

    lax.fori_loop(0, n_chunks, body, 0)

  return gather_kernel


_gather = _make_gather(4096 * 50, DIM, 640)


def kernel(input, table):
  idx = input.reshape(-1).astype(jnp.int32)
  out = _gather(table, idx)
  return out.reshape(input.shape + (table.shape[1],))


# speedup vs baseline: 3.3442x; 1.0152x over previous
import functools

import jax
import jax.numpy as jnp
from jax import lax
from jax.experimental import pallas as pl
from jax.experimental.pallas import tpu as pltpu
from jax.experimental.pallas import tpu_sc as plsc

DIM = 128
NUM_CORES = 2
NUM_SUBCORES = 16
NUM_WORKERS = NUM_CORES * NUM_SUBCORES


def _make_gather(batch: int, dim: int, chunk: int):
  assert batch % NUM_WORKERS == 0
  b_per_w = batch // NUM_WORKERS
  assert b_per_w % (2 * chunk) == 0 and chunk % 8 == 0
  n_pairs = b_per_w // (2 * chunk)

  mesh = plsc.VectorSubcoreMesh(core_axis_name="c", subcore_axis_name="s")

  @functools.partial(
      pl.kernel,
      mesh=mesh,
      out_type=jax.ShapeDtypeStruct((batch, dim), jnp.float32),
      scratch_types=[
          pltpu.VMEM((b_per_w,), jnp.int32),
          pltpu.VMEM((chunk, dim), jnp.float32),
          pltpu.VMEM((chunk, dim), jnp.float32),
          pltpu.SemaphoreType.DMA,
          pltpu.SemaphoreType.DMA,
      ],
  )
  def gather_kernel(table_hbm, idx_hbm, out_hbm, idx_v, buf0, buf1, sem0,
                    sem1):
    wid = lax.axis_index("s") * NUM_CORES + lax.axis_index("c")
    base = wid * b_per_w
    pltpu.sync_copy(idx_hbm.at[pl.ds(base, b_per_w)], idx_v)

    def gather_start(off, buf, sem):
      pltpu.async_copy(table_hbm.at[idx_v.at[pl.ds(off, chunk)]], buf, sem)

    def gather_wait(off, buf, sem):
      pltpu.make_async_copy(
          table_hbm.at[idx_v.at[pl.ds(off, chunk)]], buf, sem
      ).wait()

    gather_start(0, buf0, sem0)

    def body(p, carry):
      off0 = 2 * p * chunk
      off1 = off0 + chunk
      gather_start(off1, buf1, sem1)
      gather_wait(off0, buf0, sem0)
      pltpu.sync_copy(buf0, out_hbm.at[pl.ds(base + off0, chunk)])

      @pl.when(p + 1 < n_pairs)
      def _():
        gather_start(off0 + 2 * chunk, buf0, sem0)

      gather_wait(off1, buf1, sem1)
      pltpu.sync_copy(buf1, out_hbm.at[pl.ds(base + off1, chunk)])
      return carry

    lax.fori_loop(0, n_pairs, body, 0)

  return gather_kernel


_gather = _make_gather(4096 * 50, DIM, 400)


def kernel(input, table):
  idx = input.reshape(-1).astype(jnp.int32)
  out = _gather(table, idx)
  return out.reshape(input.shape + (table.shape[1],))


# speedup vs baseline: 5.9089x; 1.7669x over previous
import functools

import jax
import jax.numpy as jnp
from jax import lax
from jax.experimental import pallas as pl
from jax.experimental.pallas import tpu as pltpu
from jax.experimental.pallas import tpu_sc as plsc

NUM_CORES = 2
NUM_SUBCORES = 16
NUM_WORKERS = NUM_CORES * NUM_SUBCORES


def _make_lookup(batch: int, text: int, dim: int, rows_per_chunk: int):
  assert batch % NUM_WORKERS == 0
  rows_per_w = batch // NUM_WORKERS
  assert rows_per_w % (2 * rows_per_chunk) == 0
  n_pairs = rows_per_w // (2 * rows_per_chunk)
  chunk = rows_per_chunk * text
  idx_per_w = rows_per_w * text
  assert chunk % 8 == 0

  mesh = plsc.VectorSubcoreMesh(core_axis_name="c", subcore_axis_name="s")

  @functools.partial(
      pl.kernel,
      mesh=mesh,
      out_type=jax.ShapeDtypeStruct((batch, text, dim), jnp.float32),
      scratch_types=[
          pltpu.VMEM((idx_per_w,), jnp.int32),
          pltpu.VMEM((chunk, dim), jnp.float32),
          pltpu.VMEM((chunk, dim), jnp.float32),
          pltpu.SemaphoreType.DMA,
          pltpu.SemaphoreType.DMA,
      ],
  )
  def lookup_kernel(table_hbm, idx_hbm, out_hbm, idx_v, buf0, buf1, sem0,
                    sem1):
    wid = lax.axis_index("s") * NUM_CORES + lax.axis_index("c")
    row_base = wid * rows_per_w
    pltpu.sync_copy(idx_hbm.at[pl.ds(row_base * text, idx_per_w)], idx_v)

    def gather_start(c, buf, sem):
      pltpu.async_copy(
          table_hbm.at[idx_v.at[pl.ds(c * chunk, chunk)]], buf, sem
      )

    def gather_wait(c, buf, sem):
      pltpu.make_async_copy(
          table_hbm.at[idx_v.at[pl.ds(c * chunk, chunk)]], buf, sem
      ).wait()

    def store(c, buf):
      row0 = row_base + c * rows_per_chunk
      for r in range(rows_per_chunk):
        pltpu.sync_copy(
            buf.at[pl.ds(r * text, text)], out_hbm.at[row0 + r]
        )

    gather_start(0, buf0, sem0)

    def body(p, carry):
      c0 = 2 * p
      gather_start(c0 + 1, buf1, sem1)
      gather_wait(c0, buf0, sem0)
      store(c0, buf0)

      @pl.when(p + 1 < n_pairs)
      def _():
        gather_start(c0 + 2, buf0, sem0)

      gather_wait(c0 + 1, buf1, sem1)
      store(c0 + 1, buf1)
      return carry

    lax.fori_loop(0, n_pairs, body, 0)

  return lookup_kernel


_lookup = _make_lookup(4096, 50, 128, 8)


def kernel(input, table):
  idx = input.reshape(-1).astype(jnp.int32)
  return _lookup(table, idx)
